# baseline (speedup 1.0000x reference)
import functools

import jax
import jax.numpy as jnp
from jax import lax
from jax.experimental import pallas as pl
from jax.experimental.pallas import tpu as pltpu
from jax.experimental.pallas import tpu_sc as plsc

_LANES = 16


@functools.lru_cache(maxsize=None)
def _build(batch, seq, vocab, d_model):
    info = plsc.get_sparse_core_info()
    nc, ns = info.num_cores, info.num_subcores
    nw = nc * ns
    spw = seq // nw
    cs = 8
    nchunk = spw // cs
    vecs = d_model // _LANES

    assert seq % nw == 0 and spw % cs == 0 and d_model % _LANES == 0

    mesh = plsc.VectorSubcoreMesh(core_axis_name="c", subcore_axis_name="s")

    @functools.partial(
        pl.kernel,
        mesh=mesh,
        out_type=jax.ShapeDtypeStruct((batch, seq, d_model), jnp.float32),
        scratch_types=[
            pltpu.VMEM((batch, spw), jnp.int32),
            pltpu.VMEM((2, batch * cs, d_model), jnp.float32),
            pltpu.VMEM((2, cs, d_model), jnp.float32),
            pltpu.SemaphoreType.DMA,
            pltpu.SemaphoreType.DMA,
            pltpu.SemaphoreType.DMA,
        ],
    )
    def k(ids_hbm, emb_hbm, pos_hbm, out_hbm, idx_v, emb_v, pos_v,
          sem_g, sem_p, sem_o):
        wid = lax.axis_index("s") * nc + lax.axis_index("c")
        s0 = wid * spw

        idx_cps = [
            pltpu.async_copy(ids_hbm.at[b, pl.ds(s0, spw)], idx_v.at[b],
                             sem_g)
            for b in range(batch)
        ]
        for h in idx_cps:
            h.wait()

        def start(c):
            buf = c % 2
            g = [
                pltpu.async_copy(
                    emb_hbm.at[idx_v.at[b, pl.ds(c * cs, cs)]],
                    emb_v.at[buf, pl.ds(b * cs, cs)], sem_g)
                for b in range(batch)
            ]
            p = pltpu.async_copy(pos_hbm.at[pl.ds(s0 + c * cs, cs)],
                                 pos_v.at[buf], sem_p)
            return g + [p]

        def add_rows(c):
            buf = c % 2
            ec = emb_v.at[buf]
            pc = pos_v.at[buf]

            def body(t, _):
                for j in range(vecs):
                    sl = pl.ds(j * _LANES, _LANES)
                    pv = pc[t, sl]
                    for b in range(batch):
                        plsc.addupdate(ec.at[b * cs + t, sl], pv)
                return 0

            lax.fori_loop(0, cs, body, 0)

        def store(c):
            buf = c % 2
            return [
                pltpu.async_copy(
                    emb_v.at[buf, pl.ds(b * cs, cs)],
                    out_hbm.at[b, pl.ds(s0 + c * cs, cs)], sem_o)
                for b in range(batch)
            ]

        pending = {0: start(0)}
        stores = {}
        for c in range(nchunk):
            if c + 1 < nchunk:
                if c - 1 in stores:
                    for h in stores.pop(c - 1):
                        h.wait()
                pending[c + 1] = start(c + 1)
            for h in pending.pop(c):
                h.wait()
            add_rows(c)
            stores[c] = store(c)
        for c in sorted(stores):
            for h in stores[c]:
                h.wait()

    return k


def kernel(inputs, emb_table, pos_table):
    batch, seq = inputs.shape
    vocab, d_model = emb_table.shape
    return _build(batch, seq, vocab, d_model)(inputs, emb_table, pos_table)


# speedup vs baseline: 1.0615x; 1.0615x over previous
import functools

import jax
import jax.numpy as jnp
from jax import lax
from jax.experimental import pallas as pl
from jax.experimental.pallas import tpu as pltpu
from jax.experimental.pallas import tpu_sc as plsc

_LANES = 16


@functools.lru_cache(maxsize=None)
def _build(batch, seq, vocab, d_model):
    info = plsc.get_sparse_core_info()
    nc, ns = info.num_cores, info.num_subcores
    nw = nc * ns
    spw = seq // nw
    cs = 16
    nchunk = spw // cs
    vecs = d_model // _LANES

    assert seq % nw == 0 and spw % cs == 0 and d_model % _LANES == 0

    mesh = plsc.VectorSubcoreMesh(core_axis_name="c", subcore_axis_name="s")

    @functools.partial(
        pl.kernel,
        mesh=mesh,
        out_type=jax.ShapeDtypeStruct((batch, seq, d_model), jnp.float32),
        scratch_types=[
            pltpu.VMEM((batch, spw), jnp.int32),
            pltpu.VMEM((2, batch * cs, d_model), jnp.float32),
            pltpu.VMEM((2, cs, d_model), jnp.float32),
            pltpu.SemaphoreType.DMA,
            pltpu.SemaphoreType.DMA,
            pltpu.SemaphoreType.DMA,
        ],
    )
    def k(ids_hbm, emb_hbm, pos_hbm, out_hbm, idx_v, emb_v, pos_v,
          sem_g, sem_p, sem_o):
        wid = lax.axis_index("s") * nc + lax.axis_index("c")
        s0 = wid * spw

        idx_cps = [
            pltpu.async_copy(ids_hbm.at[b, pl.ds(s0, spw)], idx_v.at[b],
                             sem_g)
            for b in range(batch)
        ]
        for h in idx_cps:
            h.wait()

        def start(c):
            buf = c % 2
            g = [
                pltpu.async_copy(
                    emb_hbm.at[idx_v.at[b, pl.ds(c * cs, cs)]],
                    emb_v.at[buf, pl.ds(b * cs, cs)], sem_g)
                for b in range(batch)
            ]
            p = pltpu.async_copy(pos_hbm.at[pl.ds(s0 + c * cs, cs)],
                                 pos_v.at[buf], sem_p)
            return g + [p]

        def add_rows(c):
            buf = c % 2
            ec = emb_v.at[buf]
            pc = pos_v.at[buf]

            def body(t, _):
                for j in range(vecs):
                    sl = pl.ds(j * _LANES, _LANES)
                    pv = pc[t, sl]
                    for b in range(batch):
                        plsc.addupdate(ec.at[b * cs + t, sl], pv)
                return 0

            lax.fori_loop(0, cs, body, 0)

        def store(c):
            buf = c % 2
            return [
                pltpu.async_copy(
                    emb_v.at[buf, pl.ds(b * cs, cs)],
                    out_hbm.at[b, pl.ds(s0 + c * cs, cs)], sem_o)
                for b in range(batch)
            ]

        pending = {0: start(0)}
        stores = {}
        for c in range(nchunk):
            if c + 1 < nchunk:
                if c - 1 in stores:
                    for h in stores.pop(c - 1):
                        h.wait()
                pending[c + 1] = start(c + 1)
            for h in pending.pop(c):
                h.wait()
            add_rows(c)
            stores[c] = store(c)
        for c in sorted(stores):
            for h in stores[c]:
                h.wait()

    return k


def kernel(inputs, emb_table, pos_table):
    batch, seq = inputs.shape
    vocab, d_model = emb_table.shape
    return _build(batch, seq, vocab, d_model)(inputs, emb_table, pos_table)


# speedup vs baseline: 1.0635x; 1.0019x over previous
import functools

import jax
import jax.numpy as jnp
from jax import lax
from jax.experimental import pallas as pl
from jax.experimental.pallas import tpu as pltpu
from jax.experimental.pallas import tpu_sc as plsc

_LANES = 16


@functools.lru_cache(maxsize=None)
def _build(batch, seq, vocab, d_model):
    info = plsc.get_sparse_core_info()
    nc, ns = info.num_cores, info.num_subcores
    nw = nc * ns
    spw = seq // nw
    cmax = 16
    chunks = (8, 16, 16, 16, 8)
    offs = [sum(chunks[:i]) for i in range(len(chunks))]
    vecs = d_model // _LANES

    assert sum(chunks) == spw and max(chunks) == cmax
    assert seq % nw == 0 and d_model % _LANES == 0

    mesh = plsc.VectorSubcoreMesh(core_axis_name="c", subcore_axis_name="s")

    @functools.partial(
        pl.kernel,
        mesh=mesh,
        out_type=jax.ShapeDtypeStruct((batch, seq, d_model), jnp.float32),
        scratch_types=[
            pltpu.VMEM((batch, spw), jnp.int32),
            pltpu.VMEM((2, batch * cmax, d_model), jnp.float32),
            pltpu.VMEM((2, cmax, d_model), jnp.float32),
            pltpu.SemaphoreType.DMA,
            pltpu.SemaphoreType.DMA,
            pltpu.SemaphoreType.DMA,
        ],
    )
    def k(ids_hbm, emb_hbm, pos_hbm, out_hbm, idx_v, emb_v, pos_v,
          sem_g, sem_p, sem_o):
        wid = lax.axis_index("s") * nc + lax.axis_index("c")
        s0 = wid * spw

        pos0 = pltpu.async_copy(pos_hbm.at[pl.ds(s0, chunks[0])],
                                pos_v.at[0, pl.ds(0, chunks[0])], sem_p)
        idx_cps = [
            pltpu.async_copy(ids_hbm.at[b, pl.ds(s0, spw)], idx_v.at[b],
                             sem_g)
            for b in range(batch)
        ]
        for h in idx_cps:
            h.wait()

        def start(c, with_pos=True):
            cs = chunks[c]
            buf = c % 2
            g = [
                pltpu.async_copy(
                    emb_hbm.at[idx_v.at[b, pl.ds(offs[c], cs)]],
                    emb_v.at[buf, pl.ds(b * cs, cs)], sem_g)
                for b in range(batch)
            ]
            if with_pos:
                g.append(
                    pltpu.async_copy(pos_hbm.at[pl.ds(s0 + offs[c], cs)],
                                     pos_v.at[buf, pl.ds(0, cs)], sem_p))
            return g

        def add_rows(c):
            cs = chunks[c]
            buf = c % 2
            ec = emb_v.at[buf]
            pc = pos_v.at[buf]

            def body(t, _):
                for j in range(vecs):
                    sl = pl.ds(j * _LANES, _LANES)
                    pv = pc[t, sl]
                    for b in range(batch):
                        plsc.addupdate(ec.at[b * cs + t, sl], pv)
                return 0

            lax.fori_loop(0, cs, body, 0)

        def store(c):
            cs = chunks[c]
            buf = c % 2
            return [
                pltpu.async_copy(
                    emb_v.at[buf, pl.ds(b * cs, cs)],
                    out_hbm.at[b, pl.ds(s0 + offs[c], cs)], sem_o)
                for b in range(batch)
            ]

        nchunk = len(chunks)
        pending = {0: start(0, with_pos=False) + [pos0]}
        stores = {}
        for c in range(nchunk):
            if c + 1 < nchunk:
                if c - 1 in stores:
                    for h in stores.pop(c - 1):
                        h.wait()
                pending[c + 1] = start(c + 1)
            for h in pending.pop(c):
                h.wait()
            add_rows(c)
            stores[c] = store(c)
        for c in sorted(stores):
            for h in stores[c]:
                h.wait()

    return k


def kernel(inputs, emb_table, pos_table):
    batch, seq = inputs.shape
    vocab, d_model = emb_table.shape
    return _build(batch, seq, vocab, d_model)(inputs, emb_table, pos_table)


# speedup vs baseline: 1.0822x; 1.0176x over previous
import functools

import jax
import jax.numpy as jnp
from jax import lax
from jax.experimental import pallas as pl
from jax.experimental.pallas import tpu as pltpu
from jax.experimental.pallas import tpu_sc as plsc

_LANES = 16


@functools.lru_cache(maxsize=None)
def _build(batch, seq, vocab, d_model):
    info = plsc.get_sparse_core_info()
    nc, ns = info.num_cores, info.num_subcores
    nw = nc * ns
    spw = seq // nw
    cs = 16
    nchunk = spw // cs
    rows = batch * cs
    vecs = d_model // _LANES

    assert seq % nw == 0 and spw % cs == 0 and cs % _LANES == 0
    assert d_model % _LANES == 0 and rows <= 128

    mesh = plsc.VectorSubcoreMesh(core_axis_name="c", subcore_axis_name="s")

    @functools.partial(
        pl.kernel,
        mesh=mesh,
        out_type=jax.ShapeDtypeStruct((batch, seq, d_model), jnp.float32),
        scratch_types=[
            pltpu.VMEM((batch, spw), jnp.int32),
            pltpu.VMEM((nchunk, rows), jnp.int32),
            pltpu.VMEM((2, rows, d_model), jnp.float32),
            pltpu.VMEM((2, cs, d_model), jnp.float32),
            pltpu.SemaphoreType.DMA,
            pltpu.SemaphoreType.DMA,
            pltpu.SemaphoreType.DMA,
        ],
    )
    def k(ids_hbm, emb_hbm, pos_hbm, out_hbm, idx_v, idx2_v, emb_v, pos_v,
          sem_g, sem_p, sem_o):
        wid = lax.axis_index("s") * nc + lax.axis_index("c")
        s0 = wid * spw

        pos0 = pltpu.async_copy(pos_hbm.at[pl.ds(s0, cs)], pos_v.at[0],
                                sem_p)
        idx_cps = [
            pltpu.async_copy(ids_hbm.at[b, pl.ds(s0, spw)], idx_v.at[b],
                             sem_g)
            for b in range(batch)
        ]
        for h in idx_cps:
            h.wait()
        for c in range(nchunk):
            for b in range(batch):
                for t in range(0, cs, _LANES):
                    idx2_v[c, pl.ds(b * cs + t, _LANES)] = (
                        idx_v[b, pl.ds(c * cs + t, _LANES)])

        def start(c, with_pos=True):
            buf = c % 2
            cps = [pltpu.async_copy(emb_hbm.at[idx2_v.at[c]], emb_v.at[buf],
                                    sem_g)]
            if with_pos:
                cps.append(
                    pltpu.async_copy(pos_hbm.at[pl.ds(s0 + c * cs, cs)],
                                     pos_v.at[buf], sem_p))
            return cps

        def add_rows(c):
            buf = c % 2
            ec = emb_v.at[buf]
            pc = pos_v.at[buf]

            def body(t, _):
                for j in range(vecs):
                    sl = pl.ds(j * _LANES, _LANES)
                    pv = pc[t, sl]
                    for b in range(batch):
                        plsc.addupdate(ec.at[b * cs + t, sl], pv)
                return 0

            lax.fori_loop(0, cs, body, 0)

        def store(c):
            buf = c % 2
            return [
                pltpu.async_copy(
                    emb_v.at[buf, pl.ds(b * cs, cs)],
                    out_hbm.at[b, pl.ds(s0 + c * cs, cs)], sem_o)
                for b in range(batch)
            ]

        pending = {0: start(0, with_pos=False) + [pos0]}
        stores = {}
        for c in range(nchunk):
            if c + 1 < nchunk:
                if c - 1 in stores:
                    for h in stores.pop(c - 1):
                        h.wait()
                pending[c + 1] = start(c + 1)
            for h in pending.pop(c):
                h.wait()
            add_rows(c)
            stores[c] = store(c)
        for c in sorted(stores):
            for h in stores[c]:
                h.wait()

    return k


def kernel(inputs, emb_table, pos_table):
    batch, seq = inputs.shape
    vocab, d_model = emb_table.shape
    return _build(batch, seq, vocab, d_model)(inputs, emb_table, pos_table)


# speedup vs baseline: 1.0849x; 1.0024x over previous
import functools

import jax
import jax.numpy as jnp
from jax import lax
from jax.experimental import pallas as pl
from jax.experimental.pallas import tpu as pltpu
from jax.experimental.pallas import tpu_sc as plsc

_LANES = 16


@functools.lru_cache(maxsize=None)
def _build(batch, seq, vocab, d_model):
    info = plsc.get_sparse_core_info()
    nc, ns = info.num_cores, info.num_subcores
    nw = nc * ns
    spw = seq // nw
    cs = 16
    nchunk = spw // cs
    rows = batch * cs
    vecs = d_model // _LANES

    assert seq % nw == 0 and spw % cs == 0 and cs % _LANES == 0
    assert d_model % _LANES == 0 and rows <= 128

    mesh = plsc.VectorSubcoreMesh(core_axis_name="c", subcore_axis_name="s")

    @functools.partial(
        pl.kernel,
        mesh=mesh,
        out_type=jax.ShapeDtypeStruct((batch, seq, d_model), jnp.float32),
        scratch_types=[
            pltpu.VMEM((batch, spw), jnp.int32),
            pltpu.VMEM((nchunk, rows), jnp.int32),
            pltpu.VMEM((2, rows, d_model), jnp.float32),
            pltpu.VMEM((2, cs, d_model), jnp.float32),
            pltpu.SemaphoreType.DMA,
            pltpu.SemaphoreType.DMA,
            pltpu.SemaphoreType.DMA,
        ],
    )
    def k(ids_hbm, emb_hbm, pos_hbm, out_hbm, idx_v, idx2_v, emb_v, pos_v,
          sem_g, sem_p, sem_o):
        wid = lax.axis_index("s") * nc + lax.axis_index("c")
        s0 = wid * spw

        pos0 = pltpu.async_copy(pos_hbm.at[pl.ds(s0, cs)], pos_v.at[0],
                                sem_p)
        idx_cps = [
            pltpu.async_copy(ids_hbm.at[b, pl.ds(s0, spw)], idx_v.at[b],
                             sem_g)
            for b in range(batch)
        ]
        for h in idx_cps:
            h.wait()
        for c in range(nchunk):
            for b in range(batch):
                for t in range(0, cs, _LANES):
                    idx2_v[c, pl.ds(b * cs + t, _LANES)] = (
                        idx_v[b, pl.ds(c * cs + t, _LANES)])

        def start(c, with_pos=True):
            buf = c % 2
            cps = [pltpu.async_copy(emb_hbm.at[idx2_v.at[c]], emb_v.at[buf],
                                    sem_g)]
            if with_pos:
                cps.append(
                    pltpu.async_copy(pos_hbm.at[pl.ds(s0 + c * cs, cs)],
                                     pos_v.at[buf], sem_p))
            return cps

        def add_rows(c, lo=0, hi=cs):
            buf = c % 2
            ec = emb_v.at[buf]
            pc = pos_v.at[buf]

            def body(t, _):
                for j in range(vecs):
                    sl = pl.ds(j * _LANES, _LANES)
                    pv = pc[t, sl]
                    for b in range(batch):
                        plsc.addupdate(ec.at[b * cs + t, sl], pv)
                return 0

            lax.fori_loop(lo, hi, body, 0)

        def store(c, lo=0, hi=cs):
            buf = c % 2
            return [
                pltpu.async_copy(
                    emb_v.at[buf, pl.ds(b * cs + lo, hi - lo)],
                    out_hbm.at[b, pl.ds(s0 + c * cs + lo, hi - lo)], sem_o)
                for b in range(batch)
            ]

        pending = {0: start(0, with_pos=False) + [pos0]}
        stores = {}
        for c in range(nchunk):
            if c + 1 < nchunk:
                if c - 1 in stores:
                    for h in stores.pop(c - 1):
                        h.wait()
                pending[c + 1] = start(c + 1)
            for h in pending.pop(c):
                h.wait()
            if c + 1 < nchunk:
                add_rows(c)
                stores[c] = store(c)
            else:
                add_rows(c, 0, cs // 2)
                stores[c] = store(c, 0, cs // 2)
                add_rows(c, cs // 2, cs)
                stores[c] += store(c, cs // 2, cs)
        for c in sorted(stores):
            for h in stores[c]:
                h.wait()

    return k


def kernel(inputs, emb_table, pos_table):
    batch, seq = inputs.shape
    vocab, d_model = emb_table.shape
    return _build(batch, seq, vocab, d_model)(inputs, emb_table, pos_table)
